# baseline (speedup 1.0000x reference)
import functools

import jax
import jax.numpy as jnp
from jax import lax
from jax.experimental import pallas as pl
from jax.experimental.pallas import tpu as pltpu
from jax.experimental.pallas import tpu_sc as plsc

_T = 128
_N = 32
_C = 3
_H = 224
_W = 224


def _src_frame(w):
    bump = lambda k: jnp.where(w >= k, jnp.int32(1), jnp.int32(0))
    return jnp.int32(4) * w + bump(6) + bump(16) + bump(26)


_CHUNK_ROWS = 56
_K = _H // _CHUNK_ROWS
_NCH = _C * _K
_NB = 8


def _sc_subsample(x):
    mesh = plsc.VectorSubcoreMesh(core_axis_name="c", subcore_axis_name="s")

    @functools.partial(
        pl.kernel,
        mesh=mesh,
        out_type=jax.ShapeDtypeStruct((_N, _C, _H, _W), jnp.float32),
        scratch_types=(
            [pltpu.VMEM((_CHUNK_ROWS, _W), jnp.float32)] * _NB
            + [pltpu.SemaphoreType.DMA] * (2 * _NB)
        ),
    )
    def body(x_hbm, out_hbm, *scratch):
        bufs = scratch[:_NB]
        gsems = scratch[_NB : 2 * _NB]
        ssems = scratch[2 * _NB :]
        w = lax.axis_index("s") * 2 + lax.axis_index("c")
        f = _src_frame(w)

        def src(i):
            c, r = divmod(i, _K)
            return x_hbm.at[f, c, pl.ds(r * _CHUNK_ROWS, _CHUNK_ROWS)]

        def dst(i):
            c, r = divmod(i, _K)
            return out_hbm.at[w, c, pl.ds(r * _CHUNK_ROWS, _CHUNK_ROWS)]

        gath = [None] * _NCH
        scat = [None] * _NCH
        for i in range(min(_NB, _NCH)):
            gath[i] = pltpu.async_copy(src(i), bufs[i], gsems[i])
        for i in range(_NCH):
            b = i % _NB
            gath[i].wait()
            scat[i] = pltpu.async_copy(bufs[b], dst(i), ssems[b])
            j = i + _NB
            if j < _NCH:
                scat[i].wait()
                gath[j] = pltpu.async_copy(src(j), bufs[b], gsems[b])
        for i in range(max(0, _NCH - _NB), _NCH):
            scat[i].wait()

    return body(x)


_IDX = [4 * w + (w >= 6) + (w >= 16) + (w >= 26) for w in range(_N)]
_NQ = 8


def _tc_subsample(x):
    def body(x_ref, o_ref, *sems):
        cps = []
        for w in range(_N):
            cp = pltpu.make_async_copy(
                x_ref.at[_IDX[w]], o_ref.at[w], sems[w % _NQ]
            )
            cp.start()
            cps.append(cp)
        for cp in cps:
            cp.wait()

    return pl.pallas_call(
        body,
        in_specs=[pl.BlockSpec(memory_space=pltpu.MemorySpace.HBM)],
        out_specs=pl.BlockSpec(memory_space=pltpu.MemorySpace.HBM),
        out_shape=jax.ShapeDtypeStruct((_N, _C, _H, _W), jnp.float32),
        scratch_shapes=[pltpu.SemaphoreType.DMA] * _NQ,
    )(x)


def kernel(x):
    return _tc_subsample(x)


# speedup vs baseline: 7.9243x; 7.9243x over previous
import functools

import jax
import jax.numpy as jnp
from jax import lax
from jax.experimental import pallas as pl
from jax.experimental.pallas import tpu as pltpu
from jax.experimental.pallas import tpu_sc as plsc

_T = 128
_N = 32
_C = 3
_H = 224
_W = 224


def _src_frame(w):
    bump = lambda k: jnp.where(w >= k, jnp.int32(1), jnp.int32(0))
    return jnp.int32(4) * w + bump(6) + bump(16) + bump(26)


_CHUNK_ROWS = 56
_K = _H // _CHUNK_ROWS
_NCH = _C * _K
_NB = 8


def _sc_subsample(x):
    mesh = plsc.VectorSubcoreMesh(core_axis_name="c", subcore_axis_name="s")

    @functools.partial(
        pl.kernel,
        mesh=mesh,
        out_type=jax.ShapeDtypeStruct((_N, _C, _H, _W), jnp.float32),
        scratch_types=(
            [pltpu.VMEM((_CHUNK_ROWS, _W), jnp.float32)] * _NB
            + [pltpu.SemaphoreType.DMA] * (2 * _NB)
        ),
    )
    def body(x_hbm, out_hbm, *scratch):
        bufs = scratch[:_NB]
        gsems = scratch[_NB : 2 * _NB]
        ssems = scratch[2 * _NB :]
        w = lax.axis_index("s") * 2 + lax.axis_index("c")
        f = _src_frame(w)

        def src(i):
            c, r = divmod(i, _K)
            return x_hbm.at[f, c, pl.ds(r * _CHUNK_ROWS, _CHUNK_ROWS)]

        def dst(i):
            c, r = divmod(i, _K)
            return out_hbm.at[w, c, pl.ds(r * _CHUNK_ROWS, _CHUNK_ROWS)]

        gath = [None] * _NCH
        scat = [None] * _NCH
        for i in range(min(_NB, _NCH)):
            gath[i] = pltpu.async_copy(src(i), bufs[i], gsems[i])
        for i in range(_NCH):
            b = i % _NB
            gath[i].wait()
            scat[i] = pltpu.async_copy(bufs[b], dst(i), ssems[b])
            j = i + _NB
            if j < _NCH:
                scat[i].wait()
                gath[j] = pltpu.async_copy(src(j), bufs[b], gsems[b])
        for i in range(max(0, _NCH - _NB), _NCH):
            scat[i].wait()

    return body(x)


_IDX = [4 * w + (w >= 6) + (w >= 16) + (w >= 26) for w in range(_N)]
_NQ = 8


def _tc_subsample(x):
    def body(x_ref, o_ref, vbuf, gsem, ssem):
        gcps = [
            pltpu.make_async_copy(x_ref.at[_IDX[w]], vbuf.at[w], gsem)
            for w in range(_N)
        ]
        for cp in gcps:
            cp.start()
        for cp in gcps:
            cp.wait()
        scps = [
            pltpu.make_async_copy(vbuf.at[w], o_ref.at[w], ssem)
            for w in range(_N)
        ]
        for cp in scps:
            cp.start()
        for cp in scps:
            cp.wait()

    return pl.pallas_call(
        body,
        in_specs=[pl.BlockSpec(memory_space=pltpu.MemorySpace.HBM)],
        out_specs=pl.BlockSpec(memory_space=pltpu.MemorySpace.HBM),
        out_shape=jax.ShapeDtypeStruct((_N, _C, _H, _W), jnp.float32),
        scratch_shapes=[
            pltpu.VMEM((_N, _C, _H, _W), jnp.float32),
            pltpu.SemaphoreType.DMA,
            pltpu.SemaphoreType.DMA,
        ],
    )(x)


def kernel(x):
    return _tc_subsample(x)


# speedup vs baseline: 7.9676x; 1.0055x over previous
import functools

import jax
import jax.numpy as jnp
from jax import lax
from jax.experimental import pallas as pl
from jax.experimental.pallas import tpu as pltpu
from jax.experimental.pallas import tpu_sc as plsc

_T = 128
_N = 32
_C = 3
_H = 224
_W = 224


def _src_frame(w):
    bump = lambda k: jnp.where(w >= k, jnp.int32(1), jnp.int32(0))
    return jnp.int32(4) * w + bump(6) + bump(16) + bump(26)


_CHUNK_ROWS = 56
_K = _H // _CHUNK_ROWS
_NCH = _C * _K
_NB = 8


def _sc_subsample(x):
    mesh = plsc.VectorSubcoreMesh(core_axis_name="c", subcore_axis_name="s")

    @functools.partial(
        pl.kernel,
        mesh=mesh,
        out_type=jax.ShapeDtypeStruct((_N, _C, _H, _W), jnp.float32),
        scratch_types=(
            [pltpu.VMEM((_CHUNK_ROWS, _W), jnp.float32)] * _NB
            + [pltpu.SemaphoreType.DMA] * (2 * _NB)
        ),
    )
    def body(x_hbm, out_hbm, *scratch):
        bufs = scratch[:_NB]
        gsems = scratch[_NB : 2 * _NB]
        ssems = scratch[2 * _NB :]
        w = lax.axis_index("s") * 2 + lax.axis_index("c")
        f = _src_frame(w)

        def src(i):
            c, r = divmod(i, _K)
            return x_hbm.at[f, c, pl.ds(r * _CHUNK_ROWS, _CHUNK_ROWS)]

        def dst(i):
            c, r = divmod(i, _K)
            return out_hbm.at[w, c, pl.ds(r * _CHUNK_ROWS, _CHUNK_ROWS)]

        gath = [None] * _NCH
        scat = [None] * _NCH
        for i in range(min(_NB, _NCH)):
            gath[i] = pltpu.async_copy(src(i), bufs[i], gsems[i])
        for i in range(_NCH):
            b = i % _NB
            gath[i].wait()
            scat[i] = pltpu.async_copy(bufs[b], dst(i), ssems[b])
            j = i + _NB
            if j < _NCH:
                scat[i].wait()
                gath[j] = pltpu.async_copy(src(j), bufs[b], gsems[b])
        for i in range(max(0, _NCH - _NB), _NCH):
            scat[i].wait()

    return body(x)


_IDX = [4 * w + (w >= 6) + (w >= 16) + (w >= 26) for w in range(_N)]
_NQ = 8


def _tc_subsample(x):
    def body(x_ref, o_ref, vbuf, gsem, ssem):
        gcps = [
            pltpu.make_async_copy(x_ref.at[_IDX[w]], vbuf.at[w], gsem.at[w])
            for w in range(_N)
        ]
        for cp in gcps:
            cp.start()
        scps = []
        for w in range(_N):
            gcps[w].wait()
            scp = pltpu.make_async_copy(vbuf.at[w], o_ref.at[w], ssem.at[w])
            scp.start()
            scps.append(scp)
        for cp in scps:
            cp.wait()

    return pl.pallas_call(
        body,
        in_specs=[pl.BlockSpec(memory_space=pltpu.MemorySpace.HBM)],
        out_specs=pl.BlockSpec(memory_space=pltpu.MemorySpace.HBM),
        out_shape=jax.ShapeDtypeStruct((_N, _C, _H, _W), jnp.float32),
        scratch_shapes=[
            pltpu.VMEM((_N, _C, _H, _W), jnp.float32),
            pltpu.SemaphoreType.DMA((_N,)),
            pltpu.SemaphoreType.DMA((_N,)),
        ],
    )(x)


def kernel(x):
    return _tc_subsample(x)
